# Initial kernel scaffold; baseline (speedup 1.0000x reference)
import jax
import jax.numpy as jnp
from jax.experimental import pallas as pl


def kernel(x, edge_index, W1, b1, W2, b2):
    raise NotImplementedError("write your pallas kernel here")


# speedup vs baseline: 8.3621x; 8.3621x over previous
import functools

import jax
import jax.numpy as jnp
from jax import lax
from jax.experimental import pallas as pl
from jax.experimental.pallas import tpu as pltpu
from jax.experimental.pallas import tpu_sc as plsc

_N = 10000
_E = 320000
_D = 128
_NP = 10240
_NC = 2
_NS = 16
_NW = _NC * _NS
_EPW = _E // _NW
_CH = 80
_NCH = _EPW // _CH
_RPT = _NP // _NS


def _sc_mesh():
    return plsc.VectorSubcoreMesh(
        core_axis_name="c", subcore_axis_name="s",
        num_cores=_NC, num_subcores=_NS)


def _sc_segsum(g, src, dst, zeros_np):
    @functools.partial(
        pl.kernel,
        out_type=jax.ShapeDtypeStruct((_NC, _NP, _D), jnp.float32),
        mesh=_sc_mesh(),
        scratch_types=[
            pltpu.VMEM((_CH,), jnp.int32),
            pltpu.VMEM((_CH,), jnp.int32),
            pltpu.VMEM((_CH, _D), jnp.float32),
            pltpu.VMEM_SHARED((_NP, _D), jnp.float32),
            pltpu.SemaphoreType.DMA,
        ],
    )
    def k(g_hbm, src_hbm, dst_hbm, z_hbm, out_hbm, sidx, didx, rows, acc, sem):
        cid = lax.axis_index("c")
        sid = lax.axis_index("s")
        wid = sid * _NC + cid
        ebase = wid * _EPW
        rbase = sid * _RPT
        pltpu.sync_copy(z_hbm.at[pl.ds(rbase, _RPT)], acc.at[pl.ds(rbase, _RPT)])
        plsc.subcore_barrier()

        def body(j, carry):
            off = pl.multiple_of(ebase + j * _CH, 8)
            pltpu.sync_copy(src_hbm.at[pl.ds(off, _CH)], sidx)
            pltpu.sync_copy(dst_hbm.at[pl.ds(off, _CH)], didx)
            pltpu.async_copy(g_hbm.at[sidx], rows, sem).wait()
            pltpu.sync_copy(rows, acc.at[didx], add=True)
            return carry

        lax.fori_loop(0, _NCH, body, 0)
        plsc.subcore_barrier()
        pltpu.sync_copy(acc.at[pl.ds(rbase, _RPT)],
                        out_hbm.at[cid, pl.ds(rbase, _RPT)])

    return k(g, src, dst, zeros_np)


def _sc_degree(dst, zeros_np):
    @functools.partial(
        pl.kernel,
        out_type=jax.ShapeDtypeStruct((_NC, _NP, _D), jnp.float32),
        mesh=_sc_mesh(),
        scratch_types=[
            pltpu.VMEM((_CH,), jnp.int32),
            pltpu.VMEM((_CH, _D), jnp.float32),
            pltpu.VMEM_SHARED((_NP, _D), jnp.float32),
        ],
    )
    def k(dst_hbm, z_hbm, out_hbm, didx, ones_v, acc):
        cid = lax.axis_index("c")
        sid = lax.axis_index("s")
        wid = sid * _NC + cid
        ebase = wid * _EPW
        rbase = sid * _RPT
        pltpu.sync_copy(z_hbm.at[pl.ds(rbase, _RPT)], acc.at[pl.ds(rbase, _RPT)])

        one = jnp.full((16,), 1.0, jnp.float32)

        def fill(i, carry):
            for j in range(_D // 16):
                ones_v[i, pl.ds(j * 16, 16)] = one
            return carry

        lax.fori_loop(0, _CH, fill, 0)
        plsc.subcore_barrier()

        def body(j, carry):
            off = pl.multiple_of(ebase + j * _CH, 8)
            pltpu.sync_copy(dst_hbm.at[pl.ds(off, _CH)], didx)
            pltpu.sync_copy(ones_v, acc.at[didx], add=True)
            return carry

        lax.fori_loop(0, _NCH, body, 0)
        plsc.subcore_barrier()
        pltpu.sync_copy(acc.at[pl.ds(rbase, _RPT)],
                        out_hbm.at[cid, pl.ds(rbase, _RPT)])

    return k(dst, zeros_np)


def _dis(dp_ref):
    deg = dp_ref[0] + dp_ref[1]
    pos = deg > 0
    return jnp.where(pos, lax.rsqrt(jnp.where(pos, deg, 1.0)), 0.0)


def _tc_scale_matmul(x, w, degp):
    def body(x_ref, w_ref, dp_ref, g_ref):
        g_ref[...] = _dis(dp_ref) * jnp.dot(
            x_ref[...], w_ref[...], preferred_element_type=jnp.float32)

    return pl.pallas_call(
        body,
        grid=(_NP // 128,),
        in_specs=[
            pl.BlockSpec((128, _D), lambda i: (i, 0)),
            pl.BlockSpec((_D, _D), lambda i: (0, 0)),
            pl.BlockSpec((_NC, 128, _D), lambda i: (0, i, 0)),
        ],
        out_specs=pl.BlockSpec((128, _D), lambda i: (i, 0)),
        out_shape=jax.ShapeDtypeStruct((_NP, _D), jnp.float32),
    )(x, w, degp)


def _tc_mid(p, degp, b, w):
    def body(p_ref, dp_ref, b_ref, w_ref, h_ref, g_ref):
        dis = _dis(dp_ref)
        t = dis * (p_ref[0] + p_ref[1]) + b_ref[...]
        n = jnp.sqrt(jnp.sum(t * t, axis=-1, keepdims=True))
        h = t / jnp.maximum(n, 1e-12)
        h_ref[...] = h
        g_ref[...] = dis * jnp.dot(
            h, w_ref[...], preferred_element_type=jnp.float32)

    return pl.pallas_call(
        body,
        grid=(_NP // 128,),
        in_specs=[
            pl.BlockSpec((_NC, 128, _D), lambda i: (0, i, 0)),
            pl.BlockSpec((_NC, 128, _D), lambda i: (0, i, 0)),
            pl.BlockSpec((1, _D), lambda i: (0, 0)),
            pl.BlockSpec((_D, _D), lambda i: (0, 0)),
        ],
        out_specs=[
            pl.BlockSpec((128, _D), lambda i: (i, 0)),
            pl.BlockSpec((128, _D), lambda i: (i, 0)),
        ],
        out_shape=[
            jax.ShapeDtypeStruct((_NP, _D), jnp.float32),
            jax.ShapeDtypeStruct((_NP, _D), jnp.float32),
        ],
    )(p, degp, b, w)


def _tc_final(p, degp, b, x, h1):
    def body(p_ref, dp_ref, b_ref, x_ref, h1_ref, o_ref):
        t = _dis(dp_ref) * (p_ref[0] + p_ref[1]) + b_ref[...]
        n = jnp.sqrt(jnp.sum(t * t, axis=-1, keepdims=True))
        h2 = t / jnp.maximum(n, 1e-12)
        o_ref[...] = x_ref[...] + h1_ref[...] + 0.5 * h2

    return pl.pallas_call(
        body,
        grid=(_NP // 128,),
        in_specs=[
            pl.BlockSpec((_NC, 128, _D), lambda i: (0, i, 0)),
            pl.BlockSpec((_NC, 128, _D), lambda i: (0, i, 0)),
            pl.BlockSpec((1, _D), lambda i: (0, 0)),
            pl.BlockSpec((128, _D), lambda i: (i, 0)),
            pl.BlockSpec((128, _D), lambda i: (i, 0)),
        ],
        out_specs=pl.BlockSpec((128, _D), lambda i: (i, 0)),
        out_shape=jax.ShapeDtypeStruct((_NP, _D), jnp.float32),
    )(p, degp, b, x, h1)


def kernel(x, edge_index, W1, b1, W2, b2):
    src = edge_index[0].astype(jnp.int32)
    dst = edge_index[1].astype(jnp.int32)
    xp = jnp.pad(x, ((0, _NP - _N), (0, 0)))
    zeros_np = jnp.zeros((_NP, _D), jnp.float32)
    b1r = b1.reshape(1, _D)
    b2r = b2.reshape(1, _D)

    degp = _sc_degree(dst, zeros_np)
    g1 = _tc_scale_matmul(xp, W1, degp)
    p1 = _sc_segsum(g1, src, dst, zeros_np)
    h1, g2 = _tc_mid(p1, degp, b1r, W2)
    p2 = _sc_segsum(g2, src, dst, zeros_np)
    out = _tc_final(p2, degp, b2r, xp, h1)
    return out[:_N]


# speedup vs baseline: 12.2793x; 1.4684x over previous
import functools

import jax
import jax.numpy as jnp
from jax import lax
from jax.experimental import pallas as pl
from jax.experimental.pallas import tpu as pltpu
from jax.experimental.pallas import tpu_sc as plsc

_N = 10000
_E = 320000
_D = 128
_NP = 10240
_NC = 2
_NS = 16
_NW = _NC * _NS
_EPW = _E // _NW
_CH = 80
_NCH = _EPW // _CH
_RPT = _NP // _NS


def _sc_mesh():
    return plsc.VectorSubcoreMesh(
        core_axis_name="c", subcore_axis_name="s",
        num_cores=_NC, num_subcores=_NS)


def _sc_segsum(g, srcp, dstp, zeros_np):
    @functools.partial(
        pl.kernel,
        out_type=jax.ShapeDtypeStruct((_NC, _NP, _D), jnp.float32),
        mesh=_sc_mesh(),
        scratch_types=[
            pltpu.VMEM((_CH,), jnp.int32),
            pltpu.VMEM((_CH,), jnp.int32),
            pltpu.VMEM((_CH,), jnp.int32),
            pltpu.VMEM((_CH,), jnp.int32),
            pltpu.VMEM((_CH, _D), jnp.float32),
            pltpu.VMEM((_CH, _D), jnp.float32),
            pltpu.VMEM_SHARED((_NP, _D), jnp.float32),
            pltpu.SemaphoreType.DMA,
            pltpu.SemaphoreType.DMA,
            pltpu.SemaphoreType.DMA,
            pltpu.SemaphoreType.DMA,
        ],
    )
    def k(g_hbm, src_hbm, dst_hbm, z_hbm, out_hbm,
          sidx_a, didx_a, sidx_b, didx_b, rows_a, rows_b, acc,
          gsem_a, gsem_b, isem, zsem):
        cid = lax.axis_index("c")
        sid = lax.axis_index("s")
        wid = sid * _NC + cid
        ebase = wid * _EPW
        rbase = sid * _RPT

        def eoff(j):
            return pl.multiple_of(ebase + j * _CH, 8)

        zcp = pltpu.async_copy(z_hbm.at[pl.ds(rbase, _RPT)],
                               acc.at[pl.ds(rbase, _RPT)], zsem)
        pltpu.sync_copy(src_hbm.at[pl.ds(eoff(0), _CH)], sidx_a)
        pltpu.sync_copy(dst_hbm.at[pl.ds(eoff(0), _CH)], didx_a)
        pltpu.sync_copy(src_hbm.at[pl.ds(eoff(1), _CH)], sidx_b)
        pltpu.sync_copy(dst_hbm.at[pl.ds(eoff(1), _CH)], didx_b)
        zcp.wait()
        plsc.subcore_barrier()

        pltpu.async_copy(g_hbm.at[sidx_a], rows_a, gsem_a)

        def wait_gather(sidx, rows, sem):
            pltpu.make_async_copy(g_hbm.at[sidx], rows, sem).wait()

        def wait_idx():
            pltpu.make_async_copy(src_hbm.at[pl.ds(0, _CH)], sidx_a, isem).wait()

        def body(it, carry):
            a = 2 * it
            pltpu.async_copy(g_hbm.at[sidx_b], rows_b, gsem_b)
            wait_gather(sidx_a, rows_a, gsem_a)
            pltpu.sync_copy(rows_a, acc.at[didx_a], add=True)
            pltpu.async_copy(src_hbm.at[pl.ds(eoff(a + 2), _CH)], sidx_a, isem)
            pltpu.async_copy(dst_hbm.at[pl.ds(eoff(a + 2), _CH)], didx_a, isem)
            wait_gather(sidx_b, rows_b, gsem_b)
            pltpu.sync_copy(rows_b, acc.at[didx_b], add=True)
            pltpu.async_copy(src_hbm.at[pl.ds(eoff(a + 3), _CH)], sidx_b, isem)
            pltpu.async_copy(dst_hbm.at[pl.ds(eoff(a + 3), _CH)], didx_b, isem)
            for _ in range(4):
                wait_idx()
            pltpu.async_copy(g_hbm.at[sidx_a], rows_a, gsem_a)
            return carry

        lax.fori_loop(0, (_NCH - 1) // 2, body, 0)
        wait_gather(sidx_a, rows_a, gsem_a)
        pltpu.sync_copy(rows_a, acc.at[didx_a], add=True)
        plsc.subcore_barrier()
        pltpu.sync_copy(acc.at[pl.ds(rbase, _RPT)],
                        out_hbm.at[cid, pl.ds(rbase, _RPT)])

    return k(g, srcp, dstp, zeros_np)


def _sc_degree(dst, zeros_np):
    @functools.partial(
        pl.kernel,
        out_type=jax.ShapeDtypeStruct((_NC, _NP, _D), jnp.float32),
        mesh=_sc_mesh(),
        scratch_types=[
            pltpu.VMEM((_CH,), jnp.int32),
            pltpu.VMEM((_CH,), jnp.int32),
            pltpu.VMEM((_CH, _D), jnp.float32),
            pltpu.VMEM_SHARED((_NP, _D), jnp.float32),
            pltpu.SemaphoreType.DMA,
            pltpu.SemaphoreType.DMA,
        ],
    )
    def k(dst_hbm, z_hbm, out_hbm, didx_a, didx_b, ones_v, acc, isem, zsem):
        cid = lax.axis_index("c")
        sid = lax.axis_index("s")
        wid = sid * _NC + cid
        ebase = wid * _EPW
        rbase = sid * _RPT

        def eoff(j):
            return pl.multiple_of(ebase + j * _CH, 8)

        zcp = pltpu.async_copy(z_hbm.at[pl.ds(rbase, _RPT)],
                               acc.at[pl.ds(rbase, _RPT)], zsem)
        one = jnp.full((16,), 1.0, jnp.float32)

        def fill_one(i, carry):
            for j in range(_D // 16):
                ones_v[i, pl.ds(j * 16, 16)] = one
            return carry

        lax.fori_loop(0, _CH, fill_one, 0)
        pltpu.sync_copy(dst_hbm.at[pl.ds(eoff(0), _CH)], didx_a)
        zcp.wait()
        plsc.subcore_barrier()

        def wait_idx(didx):
            pltpu.make_async_copy(dst_hbm.at[pl.ds(0, _CH)], didx, isem).wait()

        def body(it, carry):
            a = 2 * it
            pltpu.async_copy(dst_hbm.at[pl.ds(eoff(a + 1), _CH)], didx_b, isem)
            pltpu.sync_copy(ones_v, acc.at[didx_a], add=True)
            wait_idx(didx_b)
            pltpu.async_copy(dst_hbm.at[pl.ds(eoff(a + 2), _CH)], didx_a, isem)
            pltpu.sync_copy(ones_v, acc.at[didx_b], add=True)
            wait_idx(didx_a)
            return carry

        lax.fori_loop(0, (_NCH - 1) // 2, body, 0)
        pltpu.sync_copy(ones_v, acc.at[didx_a], add=True)
        plsc.subcore_barrier()
        pltpu.sync_copy(acc.at[pl.ds(rbase, _RPT)],
                        out_hbm.at[cid, pl.ds(rbase, _RPT)])

    return k(dst, zeros_np)


def _dis(dp_ref):
    deg = dp_ref[0] + dp_ref[1]
    pos = deg > 0
    return jnp.where(pos, lax.rsqrt(jnp.where(pos, deg, 1.0)), 0.0)


def _tc_scale_matmul(x, w, degp):
    def body(x_ref, w_ref, dp_ref, g_ref):
        g_ref[...] = _dis(dp_ref) * jnp.dot(
            x_ref[...], w_ref[...], preferred_element_type=jnp.float32)

    return pl.pallas_call(
        body,
        grid=(_NP // 128,),
        in_specs=[
            pl.BlockSpec((128, _D), lambda i: (i, 0)),
            pl.BlockSpec((_D, _D), lambda i: (0, 0)),
            pl.BlockSpec((_NC, 128, _D), lambda i: (0, i, 0)),
        ],
        out_specs=pl.BlockSpec((128, _D), lambda i: (i, 0)),
        out_shape=jax.ShapeDtypeStruct((_NP, _D), jnp.float32),
    )(x, w, degp)


def _tc_mid(p, degp, b, w):
    def body(p_ref, dp_ref, b_ref, w_ref, h_ref, g_ref):
        dis = _dis(dp_ref)
        t = dis * (p_ref[0] + p_ref[1]) + b_ref[...]
        n = jnp.sqrt(jnp.sum(t * t, axis=-1, keepdims=True))
        h = t / jnp.maximum(n, 1e-12)
        h_ref[...] = h
        g_ref[...] = dis * jnp.dot(
            h, w_ref[...], preferred_element_type=jnp.float32)

    return pl.pallas_call(
        body,
        grid=(_NP // 128,),
        in_specs=[
            pl.BlockSpec((_NC, 128, _D), lambda i: (0, i, 0)),
            pl.BlockSpec((_NC, 128, _D), lambda i: (0, i, 0)),
            pl.BlockSpec((1, _D), lambda i: (0, 0)),
            pl.BlockSpec((_D, _D), lambda i: (0, 0)),
        ],
        out_specs=[
            pl.BlockSpec((128, _D), lambda i: (i, 0)),
            pl.BlockSpec((128, _D), lambda i: (i, 0)),
        ],
        out_shape=[
            jax.ShapeDtypeStruct((_NP, _D), jnp.float32),
            jax.ShapeDtypeStruct((_NP, _D), jnp.float32),
        ],
    )(p, degp, b, w)


def _tc_final(p, degp, b, x, h1):
    def body(p_ref, dp_ref, b_ref, x_ref, h1_ref, o_ref):
        t = _dis(dp_ref) * (p_ref[0] + p_ref[1]) + b_ref[...]
        n = jnp.sqrt(jnp.sum(t * t, axis=-1, keepdims=True))
        h2 = t / jnp.maximum(n, 1e-12)
        o_ref[...] = x_ref[...] + h1_ref[...] + 0.5 * h2

    return pl.pallas_call(
        body,
        grid=(_NP // 128,),
        in_specs=[
            pl.BlockSpec((_NC, 128, _D), lambda i: (0, i, 0)),
            pl.BlockSpec((_NC, 128, _D), lambda i: (0, i, 0)),
            pl.BlockSpec((1, _D), lambda i: (0, 0)),
            pl.BlockSpec((128, _D), lambda i: (i, 0)),
            pl.BlockSpec((128, _D), lambda i: (i, 0)),
        ],
        out_specs=pl.BlockSpec((128, _D), lambda i: (i, 0)),
        out_shape=jax.ShapeDtypeStruct((_NP, _D), jnp.float32),
    )(p, degp, b, x, h1)


def kernel(x, edge_index, W1, b1, W2, b2):
    src = edge_index[0].astype(jnp.int32)
    dst = edge_index[1].astype(jnp.int32)
    srcp = jnp.pad(src, (0, 2 * _CH))
    dstp = jnp.pad(dst, (0, 2 * _CH))
    xp = jnp.pad(x, ((0, _NP - _N), (0, 0)))
    zeros_np = jnp.zeros((_NP, _D), jnp.float32)
    b1r = b1.reshape(1, _D)
    b2r = b2.reshape(1, _D)

    degp = _sc_degree(dst, zeros_np)
    g1 = _tc_scale_matmul(xp, W1, degp)
    p1 = _sc_segsum(g1, srcp, dstp, zeros_np)
    h1, g2 = _tc_mid(p1, degp, b1r, W2)
    p2 = _sc_segsum(g2, srcp, dstp, zeros_np)
    out = _tc_final(p2, degp, b2r, xp, h1)
    return out[:_N]


# speedup vs baseline: 13.7045x; 1.1161x over previous
import functools

import jax
import jax.numpy as jnp
from jax import lax
from jax.experimental import pallas as pl
from jax.experimental.pallas import tpu as pltpu
from jax.experimental.pallas import tpu_sc as plsc

_N = 10000
_E = 320000
_D = 128
_NP = 10240
_NC = 2
_NS = 16
_NW = _NC * _NS
_EPW = _E // _NW
_CH = 80
_NCH = _EPW // _CH
_RPT = _NP // _NS


def _sc_mesh():
    return plsc.VectorSubcoreMesh(
        core_axis_name="c", subcore_axis_name="s",
        num_cores=_NC, num_subcores=_NS)


def _sc_segsum(g, srcp, dstp, zeros_np):
    @functools.partial(
        pl.kernel,
        out_type=jax.ShapeDtypeStruct((_NC, _NP, _D), jnp.float32),
        mesh=_sc_mesh(),
        scratch_types=[
            [pltpu.VMEM((_CH,), jnp.int32)] * 4,
            [pltpu.VMEM((_CH,), jnp.int32)] * 4,
            [pltpu.VMEM((_CH, _D), jnp.float32)] * 2,
            pltpu.VMEM_SHARED((_NP, _D), jnp.float32),
            [pltpu.SemaphoreType.DMA] * 2,
            [pltpu.SemaphoreType.DMA] * 2,
            pltpu.SemaphoreType.DMA,
            pltpu.SemaphoreType.DMA,
        ],
    )
    def k(g_hbm, src_hbm, dst_hbm, z_hbm, out_hbm,
          sidx, didx, rows, acc, gsem, ssem, isem, zsem):
        cid = lax.axis_index("c")
        sid = lax.axis_index("s")
        wid = sid * _NC + cid
        ebase = wid * _EPW
        rbase = sid * _RPT

        def eoff(j):
            return pl.multiple_of(ebase + j * _CH, 8)

        def idx_load(c, jj=None):
            off = eoff(c if jj is None else jj)
            pltpu.async_copy(src_hbm.at[pl.ds(off, _CH)], sidx[c % 4], isem)
            pltpu.async_copy(dst_hbm.at[pl.ds(off, _CH)], didx[c % 4], isem)

        def idx_wait(c):
            pltpu.make_async_copy(
                src_hbm.at[pl.ds(0, _CH)], sidx[c % 4], isem).wait()
            pltpu.make_async_copy(
                src_hbm.at[pl.ds(0, _CH)], didx[c % 4], isem).wait()

        def g_start(c):
            pltpu.async_copy(g_hbm.at[sidx[c % 4]], rows[c % 2], gsem[c % 2])

        def g_wait(c):
            pltpu.make_async_copy(g_hbm.at[sidx[c % 4]], rows[c % 2],
                                  gsem[c % 2]).wait()

        def s_start(c):
            pltpu.async_copy(rows[c % 2], acc.at[didx[c % 4]], ssem[c % 2],
                             add=True)

        def s_wait(c):
            pltpu.make_async_copy(rows[c % 2], acc.at[didx[c % 4]],
                                  ssem[c % 2]).wait()

        zcp = pltpu.async_copy(z_hbm.at[pl.ds(rbase, _RPT)],
                               acc.at[pl.ds(rbase, _RPT)], zsem)
        for c in range(3):
            pltpu.sync_copy(src_hbm.at[pl.ds(eoff(c), _CH)], sidx[c])
            pltpu.sync_copy(dst_hbm.at[pl.ds(eoff(c), _CH)], didx[c])
        zcp.wait()
        plsc.subcore_barrier()

        g_start(0)
        g_wait(0)
        g_start(1)
        s_start(0)
        idx_load(3)
        for c in range(1, 4):
            g_wait(c)
            s_wait(c - 1)
            g_start(c + 1)
            s_start(c)
            idx_wait(c + 2)
            idx_load(c + 3)

        def body(it, carry):
            base = 4 * it
            for u in range(4):
                c = base + u
                g_wait(u)
                s_wait(u - 1)
                g_start(u + 1)
                s_start(u)
                idx_wait(u + 2)
                idx_load(u + 3, jj=c + 3)
            return carry

        lax.fori_loop(1, (_NCH - 1) // 4, body, 0)
        g_wait(124)
        s_wait(123)
        pltpu.sync_copy(rows[0], acc.at[didx[0]], add=True)
        idx_wait(126)
        plsc.subcore_barrier()
        pltpu.sync_copy(acc.at[pl.ds(rbase, _RPT)],
                        out_hbm.at[cid, pl.ds(rbase, _RPT)])

    return k(g, srcp, dstp, zeros_np)


def _sc_degree(dst, zeros_np):
    @functools.partial(
        pl.kernel,
        out_type=jax.ShapeDtypeStruct((_NC, _NP, _D), jnp.float32),
        mesh=_sc_mesh(),
        scratch_types=[
            pltpu.VMEM((_CH,), jnp.int32),
            pltpu.VMEM((_CH,), jnp.int32),
            pltpu.VMEM((_CH, _D), jnp.float32),
            pltpu.VMEM_SHARED((_NP, _D), jnp.float32),
            pltpu.SemaphoreType.DMA,
            pltpu.SemaphoreType.DMA,
        ],
    )
    def k(dst_hbm, z_hbm, out_hbm, didx_a, didx_b, ones_v, acc, isem, zsem):
        cid = lax.axis_index("c")
        sid = lax.axis_index("s")
        wid = sid * _NC + cid
        ebase = wid * _EPW
        rbase = sid * _RPT

        def eoff(j):
            return pl.multiple_of(ebase + j * _CH, 8)

        zcp = pltpu.async_copy(z_hbm.at[pl.ds(rbase, _RPT)],
                               acc.at[pl.ds(rbase, _RPT)], zsem)
        one = jnp.full((16,), 1.0, jnp.float32)

        def fill_one(i, carry):
            for j in range(_D // 16):
                ones_v[i, pl.ds(j * 16, 16)] = one
            return carry

        lax.fori_loop(0, _CH, fill_one, 0)
        pltpu.sync_copy(dst_hbm.at[pl.ds(eoff(0), _CH)], didx_a)
        zcp.wait()
        plsc.subcore_barrier()

        def wait_idx(didx):
            pltpu.make_async_copy(dst_hbm.at[pl.ds(0, _CH)], didx, isem).wait()

        def body(it, carry):
            a = 2 * it
            pltpu.async_copy(dst_hbm.at[pl.ds(eoff(a + 1), _CH)], didx_b, isem)
            pltpu.sync_copy(ones_v, acc.at[didx_a], add=True)
            wait_idx(didx_b)
            pltpu.async_copy(dst_hbm.at[pl.ds(eoff(a + 2), _CH)], didx_a, isem)
            pltpu.sync_copy(ones_v, acc.at[didx_b], add=True)
            wait_idx(didx_a)
            return carry

        lax.fori_loop(0, (_NCH - 1) // 2, body, 0)
        pltpu.sync_copy(ones_v, acc.at[didx_a], add=True)
        plsc.subcore_barrier()
        pltpu.sync_copy(acc.at[pl.ds(rbase, _RPT)],
                        out_hbm.at[cid, pl.ds(rbase, _RPT)])

    return k(dst, zeros_np)


def _dis(dp_ref):
    deg = dp_ref[0] + dp_ref[1]
    pos = deg > 0
    return jnp.where(pos, lax.rsqrt(jnp.where(pos, deg, 1.0)), 0.0)


def _tc_scale_matmul(x, w, degp):
    def body(x_ref, w_ref, dp_ref, g_ref):
        g_ref[...] = _dis(dp_ref) * jnp.dot(
            x_ref[...], w_ref[...], preferred_element_type=jnp.float32)

    return pl.pallas_call(
        body,
        grid=(_NP // 128,),
        in_specs=[
            pl.BlockSpec((128, _D), lambda i: (i, 0)),
            pl.BlockSpec((_D, _D), lambda i: (0, 0)),
            pl.BlockSpec((_NC, 128, _D), lambda i: (0, i, 0)),
        ],
        out_specs=pl.BlockSpec((128, _D), lambda i: (i, 0)),
        out_shape=jax.ShapeDtypeStruct((_NP, _D), jnp.float32),
    )(x, w, degp)


def _tc_mid(p, degp, b, w):
    def body(p_ref, dp_ref, b_ref, w_ref, h_ref, g_ref):
        dis = _dis(dp_ref)
        t = dis * (p_ref[0] + p_ref[1]) + b_ref[...]
        n = jnp.sqrt(jnp.sum(t * t, axis=-1, keepdims=True))
        h = t / jnp.maximum(n, 1e-12)
        h_ref[...] = h
        g_ref[...] = dis * jnp.dot(
            h, w_ref[...], preferred_element_type=jnp.float32)

    return pl.pallas_call(
        body,
        grid=(_NP // 128,),
        in_specs=[
            pl.BlockSpec((_NC, 128, _D), lambda i: (0, i, 0)),
            pl.BlockSpec((_NC, 128, _D), lambda i: (0, i, 0)),
            pl.BlockSpec((1, _D), lambda i: (0, 0)),
            pl.BlockSpec((_D, _D), lambda i: (0, 0)),
        ],
        out_specs=[
            pl.BlockSpec((128, _D), lambda i: (i, 0)),
            pl.BlockSpec((128, _D), lambda i: (i, 0)),
        ],
        out_shape=[
            jax.ShapeDtypeStruct((_NP, _D), jnp.float32),
            jax.ShapeDtypeStruct((_NP, _D), jnp.float32),
        ],
    )(p, degp, b, w)


def _tc_final(p, degp, b, x, h1):
    def body(p_ref, dp_ref, b_ref, x_ref, h1_ref, o_ref):
        t = _dis(dp_ref) * (p_ref[0] + p_ref[1]) + b_ref[...]
        n = jnp.sqrt(jnp.sum(t * t, axis=-1, keepdims=True))
        h2 = t / jnp.maximum(n, 1e-12)
        o_ref[...] = x_ref[...] + h1_ref[...] + 0.5 * h2

    return pl.pallas_call(
        body,
        grid=(_NP // 128,),
        in_specs=[
            pl.BlockSpec((_NC, 128, _D), lambda i: (0, i, 0)),
            pl.BlockSpec((_NC, 128, _D), lambda i: (0, i, 0)),
            pl.BlockSpec((1, _D), lambda i: (0, 0)),
            pl.BlockSpec((128, _D), lambda i: (i, 0)),
            pl.BlockSpec((128, _D), lambda i: (i, 0)),
        ],
        out_specs=pl.BlockSpec((128, _D), lambda i: (i, 0)),
        out_shape=jax.ShapeDtypeStruct((_NP, _D), jnp.float32),
    )(p, degp, b, x, h1)


def kernel(x, edge_index, W1, b1, W2, b2):
    src = edge_index[0].astype(jnp.int32)
    dst = edge_index[1].astype(jnp.int32)
    srcp = jnp.pad(src, (0, 2 * _CH))
    dstp = jnp.pad(dst, (0, 2 * _CH))
    xp = jnp.pad(x, ((0, _NP - _N), (0, 0)))
    zeros_np = jnp.zeros((_NP, _D), jnp.float32)
    b1r = b1.reshape(1, _D)
    b2r = b2.reshape(1, _D)

    degp = _sc_degree(dst, zeros_np)
    g1 = _tc_scale_matmul(xp, W1, degp)
    p1 = _sc_segsum(g1, srcp, dstp, zeros_np)
    h1, g2 = _tc_mid(p1, degp, b1r, W2)
    p2 = _sc_segsum(g2, srcp, dstp, zeros_np)
    out = _tc_final(p2, degp, b2r, xp, h1)
    return out[:_N]


# speedup vs baseline: 13.9012x; 1.0144x over previous
import functools

import jax
import jax.numpy as jnp
from jax import lax
from jax.experimental import pallas as pl
from jax.experimental.pallas import tpu as pltpu
from jax.experimental.pallas import tpu_sc as plsc

_N = 10000
_E = 320000
_D = 128
_NP = 10240
_NC = 2
_NS = 16
_NW = _NC * _NS
_EPW = _E // _NW
_CH = 80
_NCH = _EPW // _CH
_RPT = _NP // _NS


def _sc_mesh():
    return plsc.VectorSubcoreMesh(
        core_axis_name="c", subcore_axis_name="s",
        num_cores=_NC, num_subcores=_NS)


def _sc_segsum(g, srcp, dstp, zeros_np):
    @functools.partial(
        pl.kernel,
        out_type=jax.ShapeDtypeStruct((_NC, _NP, _D), jnp.float32),
        mesh=_sc_mesh(),
        scratch_types=[
            [pltpu.VMEM((_CH,), jnp.int32)] * 4,
            [pltpu.VMEM((_CH,), jnp.int32)] * 4,
            [pltpu.VMEM((_CH, _D), jnp.float32)] * 2,
            pltpu.VMEM_SHARED((_NP, _D), jnp.float32),
            [pltpu.SemaphoreType.DMA] * 2,
            [pltpu.SemaphoreType.DMA] * 2,
            pltpu.SemaphoreType.DMA,
            pltpu.SemaphoreType.DMA,
        ],
    )
    def k(g_hbm, src_hbm, dst_hbm, z_hbm, out_hbm,
          sidx, didx, rows, acc, gsem, ssem, isem, zsem):
        cid = lax.axis_index("c")
        sid = lax.axis_index("s")
        wid = sid * _NC + cid
        ebase = wid * _EPW
        rbase = sid * _RPT

        def eoff(j):
            return pl.multiple_of(ebase + j * _CH, 8)

        def idx_load(c, jj=None):
            off = eoff(c if jj is None else jj)
            pltpu.async_copy(src_hbm.at[pl.ds(off, _CH)], sidx[c % 4], isem)
            pltpu.async_copy(dst_hbm.at[pl.ds(off, _CH)], didx[c % 4], isem)

        def idx_wait(c):
            pltpu.make_async_copy(
                src_hbm.at[pl.ds(0, _CH)], sidx[c % 4], isem).wait()
            pltpu.make_async_copy(
                src_hbm.at[pl.ds(0, _CH)], didx[c % 4], isem).wait()

        def g_start(c):
            pltpu.async_copy(g_hbm.at[sidx[c % 4]], rows[c % 2], gsem[c % 2])

        def g_wait(c):
            pltpu.make_async_copy(g_hbm.at[sidx[c % 4]], rows[c % 2],
                                  gsem[c % 2]).wait()

        def s_start(c):
            pltpu.async_copy(rows[c % 2], acc.at[didx[c % 4]], ssem[c % 2],
                             add=True)

        def s_wait(c):
            pltpu.make_async_copy(rows[c % 2], acc.at[didx[c % 4]],
                                  ssem[c % 2]).wait()

        zcp = pltpu.async_copy(z_hbm.at[pl.ds(rbase, _RPT)],
                               acc.at[pl.ds(rbase, _RPT)], zsem)
        for c in range(3):
            pltpu.sync_copy(src_hbm.at[pl.ds(eoff(c), _CH)], sidx[c])
            pltpu.sync_copy(dst_hbm.at[pl.ds(eoff(c), _CH)], didx[c])
        zcp.wait()
        plsc.subcore_barrier()

        g_start(0)
        g_wait(0)
        g_start(1)
        s_start(0)
        idx_load(3)
        for c in range(1, 4):
            g_wait(c)
            s_wait(c - 1)
            g_start(c + 1)
            s_start(c)
            idx_wait(c + 2)
            idx_load(c + 3)

        def body(it, carry):
            base = 4 * it
            for u in range(4):
                c = base + u
                g_wait(u)
                s_wait(u - 1)
                g_start(u + 1)
                s_start(u)
                idx_wait(u + 2)
                idx_load(u + 3, jj=c + 3)
            return carry

        lax.fori_loop(1, (_NCH - 1) // 4, body, 0)
        g_wait(124)
        s_wait(123)
        pltpu.sync_copy(rows[0], acc.at[didx[0]], add=True)
        idx_wait(126)
        plsc.subcore_barrier()
        pltpu.sync_copy(acc.at[pl.ds(rbase, _RPT)],
                        out_hbm.at[cid, pl.ds(rbase, _RPT)])

    return k(g, srcp, dstp, zeros_np)


def _sc_degree(dst, zeros_np):
    @functools.partial(
        pl.kernel,
        out_type=jax.ShapeDtypeStruct((_NC, _NP, _D), jnp.float32),
        mesh=_sc_mesh(),
        scratch_types=[
            pltpu.VMEM((_CH,), jnp.int32),
            pltpu.VMEM((_CH,), jnp.int32),
            pltpu.VMEM((_CH, _D), jnp.float32),
            pltpu.VMEM_SHARED((_NP, _D), jnp.float32),
            pltpu.SemaphoreType.DMA,
            pltpu.SemaphoreType.DMA,
        ],
    )
    def k(dst_hbm, z_hbm, out_hbm, didx_a, didx_b, ones_v, acc, isem, zsem):
        cid = lax.axis_index("c")
        sid = lax.axis_index("s")
        wid = sid * _NC + cid
        ebase = wid * _EPW
        rbase = sid * _RPT

        def eoff(j):
            return pl.multiple_of(ebase + j * _CH, 8)

        zcp = pltpu.async_copy(z_hbm.at[pl.ds(rbase, _RPT)],
                               acc.at[pl.ds(rbase, _RPT)], zsem)
        one = jnp.full((16,), 1.0, jnp.float32)

        def fill_one(i, carry):
            for j in range(_D // 16):
                ones_v[i, pl.ds(j * 16, 16)] = one
            return carry

        lax.fori_loop(0, _CH, fill_one, 0)
        pltpu.sync_copy(dst_hbm.at[pl.ds(eoff(0), _CH)], didx_a)
        zcp.wait()
        plsc.subcore_barrier()

        def wait_idx(didx):
            pltpu.make_async_copy(dst_hbm.at[pl.ds(0, _CH)], didx, isem).wait()

        def body(it, carry):
            a = 2 * it
            pltpu.async_copy(dst_hbm.at[pl.ds(eoff(a + 1), _CH)], didx_b, isem)
            pltpu.sync_copy(ones_v, acc.at[didx_a], add=True)
            wait_idx(didx_b)
            pltpu.async_copy(dst_hbm.at[pl.ds(eoff(a + 2), _CH)], didx_a, isem)
            pltpu.sync_copy(ones_v, acc.at[didx_b], add=True)
            wait_idx(didx_a)
            return carry

        lax.fori_loop(0, (_NCH - 1) // 2, body, 0)
        pltpu.sync_copy(ones_v, acc.at[didx_a], add=True)
        plsc.subcore_barrier()
        pltpu.sync_copy(acc.at[pl.ds(rbase, _RPT)],
                        out_hbm.at[cid, pl.ds(rbase, _RPT)])

    return k(dst, zeros_np)


def _dis(dp_ref):
    deg = dp_ref[0] + dp_ref[1]
    pos = deg > 0
    return jnp.where(pos, lax.rsqrt(jnp.where(pos, deg, 1.0)), 0.0)


_NG = (_N + 127) // 128


def _tc_matmul(x, w):
    def body(x_ref, w_ref, o_ref):
        o_ref[...] = jnp.dot(x_ref[...], w_ref[...],
                             preferred_element_type=jnp.float32)

    return pl.pallas_call(
        body,
        grid=(_NG,),
        in_specs=[
            pl.BlockSpec((128, _D), lambda i: (i, 0)),
            pl.BlockSpec((_D, _D), lambda i: (0, 0)),
        ],
        out_specs=pl.BlockSpec((128, _D), lambda i: (i, 0)),
        out_shape=jax.ShapeDtypeStruct((_NP, _D), jnp.float32),
    )(x, w)


def _tc_scale(xw, degp):
    def body(xw_ref, dp_ref, g_ref):
        g_ref[...] = _dis(dp_ref) * xw_ref[...]

    return pl.pallas_call(
        body,
        grid=(_NG,),
        in_specs=[
            pl.BlockSpec((128, _D), lambda i: (i, 0)),
            pl.BlockSpec((_NC, 128, _D), lambda i: (0, i, 0)),
        ],
        out_specs=pl.BlockSpec((128, _D), lambda i: (i, 0)),
        out_shape=jax.ShapeDtypeStruct((_NP, _D), jnp.float32),
    )(xw, degp)


def _tc_mid(p, degp, b, w):
    def body(p_ref, dp_ref, b_ref, w_ref, h_ref, g_ref):
        dis = _dis(dp_ref)
        t = dis * (p_ref[0] + p_ref[1]) + b_ref[...]
        n = jnp.sqrt(jnp.sum(t * t, axis=-1, keepdims=True))
        h = t / jnp.maximum(n, 1e-12)
        h_ref[...] = h
        g_ref[...] = dis * jnp.dot(
            h, w_ref[...], preferred_element_type=jnp.float32)

    return pl.pallas_call(
        body,
        grid=(_NG,),
        in_specs=[
            pl.BlockSpec((_NC, 128, _D), lambda i: (0, i, 0)),
            pl.BlockSpec((_NC, 128, _D), lambda i: (0, i, 0)),
            pl.BlockSpec((1, _D), lambda i: (0, 0)),
            pl.BlockSpec((_D, _D), lambda i: (0, 0)),
        ],
        out_specs=[
            pl.BlockSpec((128, _D), lambda i: (i, 0)),
            pl.BlockSpec((128, _D), lambda i: (i, 0)),
        ],
        out_shape=[
            jax.ShapeDtypeStruct((_NP, _D), jnp.float32),
            jax.ShapeDtypeStruct((_NP, _D), jnp.float32),
        ],
    )(p, degp, b, w)


def _tc_final(p, degp, b, x, h1):
    def body(p_ref, dp_ref, b_ref, x_ref, h1_ref, o_ref):
        t = _dis(dp_ref) * (p_ref[0] + p_ref[1]) + b_ref[...]
        n = jnp.sqrt(jnp.sum(t * t, axis=-1, keepdims=True))
        h2 = t / jnp.maximum(n, 1e-12)
        o_ref[...] = x_ref[...] + h1_ref[...] + 0.5 * h2

    return pl.pallas_call(
        body,
        grid=(_NG,),
        in_specs=[
            pl.BlockSpec((_NC, 128, _D), lambda i: (0, i, 0)),
            pl.BlockSpec((_NC, 128, _D), lambda i: (0, i, 0)),
            pl.BlockSpec((1, _D), lambda i: (0, 0)),
            pl.BlockSpec((128, _D), lambda i: (i, 0)),
            pl.BlockSpec((128, _D), lambda i: (i, 0)),
        ],
        out_specs=pl.BlockSpec((128, _D), lambda i: (i, 0)),
        out_shape=jax.ShapeDtypeStruct((_N, _D), jnp.float32),
    )(p, degp, b, x, h1)


def kernel(x, edge_index, W1, b1, W2, b2):
    src = edge_index[0].astype(jnp.int32)
    dst = edge_index[1].astype(jnp.int32)
    srcp = jnp.pad(src, (0, 2 * _CH))
    dstp = jnp.pad(dst, (0, 2 * _CH))
    zeros_np = jnp.zeros((_NP, _D), jnp.float32)
    b1r = b1.reshape(1, _D)
    b2r = b2.reshape(1, _D)

    xw1 = _tc_matmul(x, W1)
    degp = _sc_degree(dst, zeros_np)
    g1 = _tc_scale(xw1, degp)
    p1 = _sc_segsum(g1, srcp, dstp, zeros_np)
    h1, g2 = _tc_mid(p1, degp, b1r, W2)
    p2 = _sc_segsum(g2, srcp, dstp, zeros_np)
    return _tc_final(p2, degp, b2r, x, h1)
